# baseline (speedup 1.0000x reference)
import jax
import jax.numpy as jnp
from jax.experimental import pallas as pl
from jax.experimental.pallas import tpu as pltpu

H = 768
FF = 2048
BB = 512


def _ln(x, g, b, eps=1e-5):
    mu = jnp.mean(x, axis=-1, keepdims=True)
    var = jnp.mean((x - mu) ** 2, axis=-1, keepdims=True)
    return (x - mu) * jax.lax.rsqrt(var + eps) * g + b


def _dot(a, b):
    return jnp.dot(a, b, preferred_element_type=jnp.float32)


def _prep_kernel(pg0, pg_last, Wv_g, bv_g, Wo_g, bo_g,
                 pl0, pl_last, Wv_l, bv_l, Wo_l, bo_l,
                 cg_out, cl_out):
    vg = _dot(pg0[...], Wv_g[...]) + bv_g[...]
    cg_out[...] = pg_last[...] + _dot(vg, Wo_g[...]) + bo_g[...]
    vl = _dot(pl0[...], Wv_l[...]) + bv_l[...]
    cl_out[...] = pl_last[...] + _dot(vl, Wo_l[...]) + bo_l[...]


def _main_kernel(z_ref,
                 cg, g_l1g, g_l1b, g_b1, g_b2, g_l2g, g_l2b,
                 cl, l_l1g, l_l1b, l_b1, l_b2, l_l2g, l_l2b,
                 d_b1, d_b2, d_lng, d_lnb, g_b,
                 gW1_h, gW2_h, lW1_h, lW2_h, dW1_h, dW2_h, gw_h,
                 out_ref,
                 gW1, gW2, lW1, lW2, dW1, dW2, gw, sems):
    i = pl.program_id(0)
    cps = [
        pltpu.make_async_copy(dW1_h, dW1, sems.at[0]),
        pltpu.make_async_copy(dW2_h, dW2, sems.at[1]),
        pltpu.make_async_copy(gw_h, gw, sems.at[2]),
        pltpu.make_async_copy(lW1_h, lW1, sems.at[3]),
        pltpu.make_async_copy(lW2_h, lW2, sems.at[4]),
        pltpu.make_async_copy(gW1_h, gW1, sems.at[5]),
        pltpu.make_async_copy(gW2_h, gW2, sems.at[6]),
    ]

    @pl.when(i == 0)
    def _():
        for c in cps:
            c.start()
        cps[0].wait()
        cps[1].wait()
        cps[2].wait()

    z = z_ref[...]

    h = jax.nn.gelu(_dot(z, dW1[...]) + d_b1[...])
    x_d = _ln(z + _dot(h, dW2[...]) + d_b2[...], d_lng[...], d_lnb[...])

    logits = _dot(z, gw[...]) + g_b[...]
    m = jnp.max(logits, axis=-1, keepdims=True)
    e = jnp.exp(logits - m)
    s = jnp.sum(e, axis=-1, keepdims=True)

    def branch(c, l1g, l1b, W1, b1, W2, b2, l2g, l2b):
        x1 = _ln(z + c[...], l1g[...], l1b[...])
        t = _dot(jnp.maximum(_dot(x1, W1[...]) + b1[...], 0.0), W2[...]) + b2[...]
        return _ln(x1 + t, l2g[...], l2b[...])

    @pl.when(i == 0)
    def _():
        cps[3].wait()
        cps[4].wait()

    x_l = branch(cl, l_l1g, l_l1b, lW1, l_b1, lW2, l_b2, l_l2g, l_l2b)

    @pl.when(i == 0)
    def _():
        cps[5].wait()
        cps[6].wait()

    x_g = branch(cg, g_l1g, g_l1b, gW1, g_b1, gW2, g_b2, g_l2g, g_l2b)

    out_ref[...] = (e[:, 0:1] * x_d + e[:, 1:2] * x_l + e[:, 2:3] * x_g) / s


@jax.jit
def kernel(z_k, params):
    B = z_k.shape[0]
    pg, plc = params['global'], params['local']
    r = lambda v: v.reshape(1, -1)

    cg, cl = pl.pallas_call(
        _prep_kernel,
        out_shape=(jax.ShapeDtypeStruct((1, H), jnp.float32),
                   jax.ShapeDtypeStruct((1, H), jnp.float32)),
    )(pg['pos'][0:1], pg['pos'][11:12], pg['Wv'], r(pg['bv']), pg['Wo'], r(pg['bo']),
      plc['pos'][0:1], plc['pos'][4:5], plc['Wv'], r(plc['bv']), plc['Wo'], r(plc['bo']))

    gW = jnp.zeros((H, 128), jnp.float32).at[:, :3].set(params['g_W'])
    gb = jnp.full((1, 128), -1e30, jnp.float32).at[0, :3].set(params['g_b'])

    w = lambda shape: pl.BlockSpec(shape, lambda i: (0, 0))
    vec = w((1, H))
    hbm = pl.BlockSpec(memory_space=pl.ANY)
    small_ops = [
        cg, r(pg['ln1_g']), r(pg['ln1_b']), r(pg['b1']), r(pg['b2']),
        r(pg['ln2_g']), r(pg['ln2_b']),
        cl, r(plc['ln1_g']), r(plc['ln1_b']), r(plc['b1']), r(plc['b2']),
        r(plc['ln2_g']), r(plc['ln2_b']),
        r(params['d_b1']), r(params['d_b2']), r(params['d_ln_g']), r(params['d_ln_b']),
        gb,
    ]
    small_specs = [vec, vec, vec, w((1, FF)), vec, vec, vec,
                   vec, vec, vec, w((1, FF)), vec, vec, vec,
                   vec, vec, vec, vec, w((1, 128))]
    big_ops = [pg['W1'], pg['W2'], plc['W1'], plc['W2'],
               params['d_W1'], params['d_W2'], gW]
    big_specs = [hbm] * 7

    f32 = jnp.float32
    out = pl.pallas_call(
        _main_kernel,
        grid=(B // BB,),
        in_specs=[pl.BlockSpec((BB, H), lambda i: (i, 0))] + small_specs + big_specs,
        out_specs=pl.BlockSpec((BB, H), lambda i: (i, 0)),
        out_shape=jax.ShapeDtypeStruct((B, H), f32),
        scratch_shapes=[
            pltpu.VMEM((H, FF), f32), pltpu.VMEM((FF, H), f32),
            pltpu.VMEM((H, FF), f32), pltpu.VMEM((FF, H), f32),
            pltpu.VMEM((H, H), f32), pltpu.VMEM((H, H), f32),
            pltpu.VMEM((H, 128), f32),
            pltpu.SemaphoreType.DMA((7,)),
        ],
        compiler_params=pltpu.CompilerParams(
            dimension_semantics=("arbitrary",)),
    )(z_k, *small_ops, *big_ops)
    return out


# speedup vs baseline: 1.2677x; 1.2677x over previous
import jax
import jax.numpy as jnp
from jax.experimental import pallas as pl
from jax.experimental.pallas import tpu as pltpu

H = 768
FF = 2048
BB = 512


def _ln(x, g, b, eps=1e-5):
    mu = jnp.mean(x, axis=-1, keepdims=True)
    var = jnp.mean((x - mu) ** 2, axis=-1, keepdims=True)
    return (x - mu) * jax.lax.rsqrt(var + eps) * g + b


def _dot(a, b):
    return jnp.dot(a, b, preferred_element_type=jnp.float32)


def _prep_kernel(pos_g, Wv_g, bv_g, Wo_g, bo_g,
                 pos_l, Wv_l, bv_l, Wo_l, bo_l,
                 cg_out, cl_out):
    vg = _dot(pos_g[0:1, :], Wv_g[...]) + bv_g[...]
    cg_out[...] = pos_g[11:12, :] + _dot(vg, Wo_g[...]) + bo_g[...]
    vl = _dot(pos_l[0:1, :], Wv_l[...]) + bv_l[...]
    cl_out[...] = pos_l[4:5, :] + _dot(vl, Wo_l[...]) + bo_l[...]


def _main_kernel(z_ref,
                 cg, g_l1g, g_l1b, g_W1, g_b1, g_W2, g_b2, g_l2g, g_l2b,
                 cl, l_l1g, l_l1b, l_W1, l_b1, l_W2, l_b2, l_l2g, l_l2b,
                 d_W1, d_b1, d_W2, d_b2, d_lng, d_lnb,
                 g_W, g_b,
                 out_ref):
    z = z_ref[...]

    def branch(c, l1g, l1b, W1, b1, W2, b2, l2g, l2b):
        x1 = _ln(z + c[...], l1g[...], l1b[...])
        t = _dot(jnp.maximum(_dot(x1, W1[...]) + b1[...], 0.0), W2[...]) + b2[...]
        return _ln(x1 + t, l2g[...], l2b[...])

    x_g = branch(cg, g_l1g, g_l1b, g_W1, g_b1, g_W2, g_b2, g_l2g, g_l2b)
    x_l = branch(cl, l_l1g, l_l1b, l_W1, l_b1, l_W2, l_b2, l_l2g, l_l2b)

    h = jax.nn.gelu(_dot(z, d_W1[...]) + d_b1[...])
    x_d = _ln(z + _dot(h, d_W2[...]) + d_b2[...], d_lng[...], d_lnb[...])

    logits = _dot(z, g_W[...]) + g_b[...]
    m = jnp.max(logits, axis=-1, keepdims=True)
    e = jnp.exp(logits - m)
    s = jnp.sum(e, axis=-1, keepdims=True)
    out_ref[...] = (e[:, 0:1] * x_d + e[:, 1:2] * x_l + e[:, 2:3] * x_g) / s


@jax.jit
def kernel(z_k, params):
    B = z_k.shape[0]
    pg, plc = params['global'], params['local']
    full = pl.BlockSpec(memory_space=pltpu.MemorySpace.VMEM)

    cg, cl = pl.pallas_call(
        _prep_kernel,
        out_shape=(jax.ShapeDtypeStruct((1, H), jnp.float32),
                   jax.ShapeDtypeStruct((1, H), jnp.float32)),
        in_specs=[full] * 10,
        out_specs=(full, full),
    )(pg['pos'], pg['Wv'], pg['bv'], pg['Wo'], pg['bo'],
      plc['pos'], plc['Wv'], plc['bv'], plc['Wo'], plc['bo'])

    w = lambda: pl.BlockSpec(None, lambda i: 0)
    operands = [
        cg, pg['ln1_g'], pg['ln1_b'], pg['W1'], pg['b1'], pg['W2'], pg['b2'],
        pg['ln2_g'], pg['ln2_b'],
        cl, plc['ln1_g'], plc['ln1_b'], plc['W1'], plc['b1'], plc['W2'], plc['b2'],
        plc['ln2_g'], plc['ln2_b'],
        params['d_W1'], params['d_b1'], params['d_W2'], params['d_b2'],
        params['d_ln_g'], params['d_ln_b'],
        params['g_W'], params['g_b'],
    ]
    specs = [full] * len(operands)

    out = pl.pallas_call(
        _main_kernel,
        grid=(B // BB,),
        in_specs=[pl.BlockSpec((BB, H), lambda i: (i, 0))] + specs,
        out_specs=pl.BlockSpec((BB, H), lambda i: (i, 0)),
        out_shape=jax.ShapeDtypeStruct((B, H), jnp.float32),
        compiler_params=pltpu.CompilerParams(
            dimension_semantics=("arbitrary",)),
    )(z_k, *operands)
    return out


# speedup vs baseline: 1.3863x; 1.0936x over previous
import jax
import jax.numpy as jnp
from jax.experimental import pallas as pl
from jax.experimental.pallas import tpu as pltpu

H = 768
FF = 2048
BB = 512


def _ln(x, g, b, eps=1e-5):
    mu = jnp.mean(x, axis=-1, keepdims=True)
    var = jnp.mean((x - mu) ** 2, axis=-1, keepdims=True)
    return (x - mu) * jax.lax.rsqrt(var + eps) * g + b


def _dot(a, b):
    return jnp.dot(a, b, preferred_element_type=jnp.float32)


def _main_kernel(z_ref,
                 g_l1g, g_l1b, g_b1, g_b2, g_l2g, g_l2b, g_bv, g_bo,
                 l_l1g, l_l1b, l_b1, l_b2, l_l2g, l_l2b, l_bv, l_bo,
                 d_b1, d_b2, d_lng, d_lnb, g_b,
                 dW1_h, dW2_h, gw_h,
                 posl_h, Wvl_h, Wol_h, lW1_h, lW2_h,
                 posg_h, Wvg_h, Wog_h, gW1_h, gW2_h,
                 out_ref,
                 dW1, dW2, gw, posl, Wvl, Wol, lW1, lW2,
                 posg, Wvg, Wog, gW1, gW2, cg_s, cl_s, sems):
    i = pl.program_id(0)
    hbm = [dW1_h, dW2_h, gw_h, posl_h, Wvl_h, Wol_h, lW1_h, lW2_h,
           posg_h, Wvg_h, Wog_h, gW1_h, gW2_h]
    vmem = [dW1, dW2, gw, posl, Wvl, Wol, lW1, lW2,
            posg, Wvg, Wog, gW1, gW2]
    cps = [pltpu.make_async_copy(h, v, sems.at[k])
           for k, (h, v) in enumerate(zip(hbm, vmem))]

    @pl.when(i == 0)
    def _():
        for c in cps:
            c.start()
        for k in (0, 1, 2):
            cps[k].wait()

    z = z_ref[...]

    h = jax.nn.gelu(_dot(z, dW1[...]) + d_b1[...])
    x_d = _ln(z + _dot(h, dW2[...]) + d_b2[...], d_lng[...], d_lnb[...])

    logits = _dot(z, gw[...]) + g_b[...]
    m = jnp.max(logits, axis=-1, keepdims=True)
    e = jnp.exp(logits - m)
    s = jnp.sum(e, axis=-1, keepdims=True)

    def branch(c_s, l1g, l1b, W1, b1, W2, b2, l2g, l2b):
        x1 = _ln(z + c_s[...], l1g[...], l1b[...])
        t = _dot(jnp.maximum(_dot(x1, W1[...]) + b1[...], 0.0), W2[...]) + b2[...]
        return _ln(x1 + t, l2g[...], l2b[...])

    @pl.when(i == 0)
    def _():
        for k in (3, 4, 5):
            cps[k].wait()
        vl = _dot(posl[0:1, :], Wvl[...]) + l_bv[...]
        cl_s[...] = posl[4:5, :] + _dot(vl, Wol[...]) + l_bo[...]
        for k in (6, 7):
            cps[k].wait()

    x_l = branch(cl_s, l_l1g, l_l1b, lW1, l_b1, lW2, l_b2, l_l2g, l_l2b)

    @pl.when(i == 0)
    def _():
        for k in (8, 9, 10):
            cps[k].wait()
        vg = _dot(posg[0:1, :], Wvg[...]) + g_bv[...]
        cg_s[...] = posg[11:12, :] + _dot(vg, Wog[...]) + g_bo[...]
        for k in (11, 12):
            cps[k].wait()

    x_g = branch(cg_s, g_l1g, g_l1b, gW1, g_b1, gW2, g_b2, g_l2g, g_l2b)

    out_ref[...] = (e[:, 0:1] * x_d + e[:, 1:2] * x_l + e[:, 2:3] * x_g) / s


@jax.jit
def kernel(z_k, params):
    B = z_k.shape[0]
    pg, plc = params['global'], params['local']
    full = pl.BlockSpec(memory_space=pltpu.MemorySpace.VMEM)
    hbm = pl.BlockSpec(memory_space=pl.ANY)

    small_ops = [
        pg['ln1_g'], pg['ln1_b'], pg['b1'], pg['b2'], pg['ln2_g'], pg['ln2_b'],
        pg['bv'], pg['bo'],
        plc['ln1_g'], plc['ln1_b'], plc['b1'], plc['b2'], plc['ln2_g'], plc['ln2_b'],
        plc['bv'], plc['bo'],
        params['d_b1'], params['d_b2'], params['d_ln_g'], params['d_ln_b'],
        params['g_b'],
    ]
    big_ops = [
        params['d_W1'], params['d_W2'], params['g_W'],
        plc['pos'], plc['Wv'], plc['Wo'], plc['W1'], plc['W2'],
        pg['pos'], pg['Wv'], pg['Wo'], pg['W1'], pg['W2'],
    ]

    f32 = jnp.float32
    vm = pltpu.VMEM
    out = pl.pallas_call(
        _main_kernel,
        grid=(B // BB,),
        in_specs=([pl.BlockSpec((BB, H), lambda i: (i, 0))]
                  + [full] * len(small_ops) + [hbm] * len(big_ops)),
        out_specs=pl.BlockSpec((BB, H), lambda i: (i, 0)),
        out_shape=jax.ShapeDtypeStruct((B, H), f32),
        scratch_shapes=[
            vm((H, H), f32), vm((H, H), f32), vm((H, 3), f32),
            vm((5, H), f32), vm((H, H), f32), vm((H, H), f32),
            vm((H, FF), f32), vm((FF, H), f32),
            vm((12, H), f32), vm((H, H), f32), vm((H, H), f32),
            vm((H, FF), f32), vm((FF, H), f32),
            vm((1, H), f32), vm((1, H), f32),
            pltpu.SemaphoreType.DMA((13,)),
        ],
        compiler_params=pltpu.CompilerParams(
            dimension_semantics=("arbitrary",)),
    )(z_k, *small_ops, *big_ops)
    return out
